# Initial kernel scaffold; baseline (speedup 1.0000x reference)
import jax
import jax.numpy as jnp
from jax.experimental import pallas as pl


def kernel(x, adapter_mapping, W, b, lora_As, lora_Bs):
    raise NotImplementedError("write your pallas kernel here")


# speedup vs baseline: 9.3258x; 9.3258x over previous
import jax
import jax.numpy as jnp
from jax.experimental import pallas as pl
from jax.experimental.pallas import tpu as pltpu

_TOKENS = 16384
_D_IN = 1024
_D_OUT = 1024
_SLOTS = 64
_R = 16
_SCALING = 2.0
_BT = 512


def _fused_body(map_ref, x_ref, wt_ref, b_ref, af_ref, bf_ref, out_ref):
    x = x_ref[...]
    base = jnp.dot(x, wt_ref[...], preferred_element_type=jnp.float32)
    h = jnp.dot(x, af_ref[...], preferred_element_type=jnp.float32)
    m = map_ref[0, 0, :]
    col_slot = jax.lax.broadcasted_iota(jnp.int32, (_BT, _SLOTS * _R), 1) // _R
    hm = jnp.where(col_slot == m[:, None], h, 0.0)
    lora = jnp.dot(hm, bf_ref[...], preferred_element_type=jnp.float32)
    out_ref[...] = base + b_ref[...] + _SCALING * lora


def kernel(x, adapter_mapping, W, b, lora_As, lora_Bs):
    n_blocks = _TOKENS // _BT
    wt = W.T
    a_flat = lora_As.transpose(1, 0, 2).reshape(_D_IN, _SLOTS * _R)
    b_flat = lora_Bs.reshape(_SLOTS * _R, _D_OUT)
    map3 = adapter_mapping.reshape(n_blocks, 1, _BT)
    b2 = b.reshape(1, _D_OUT)

    grid = (n_blocks,)
    out = pl.pallas_call(
        _fused_body,
        grid=grid,
        in_specs=[
            pl.BlockSpec((1, 1, _BT), lambda i: (i, 0, 0)),
            pl.BlockSpec((_BT, _D_IN), lambda i: (i, 0)),
            pl.BlockSpec((_D_IN, _D_OUT), lambda i: (0, 0)),
            pl.BlockSpec((1, _D_OUT), lambda i: (0, 0)),
            pl.BlockSpec((_D_IN, _SLOTS * _R), lambda i: (0, 0)),
            pl.BlockSpec((_SLOTS * _R, _D_OUT), lambda i: (0, 0)),
        ],
        out_specs=pl.BlockSpec((_BT, _D_OUT), lambda i: (i, 0)),
        out_shape=jax.ShapeDtypeStruct((_TOKENS, _D_OUT), jnp.float32),
        compiler_params=pltpu.CompilerParams(
            dimension_semantics=("arbitrary",),
        ),
    )(map3, x, wt, b2, a_flat, b_flat)
    return out


# speedup vs baseline: 9.4106x; 1.0091x over previous
import jax
import jax.numpy as jnp
from jax.experimental import pallas as pl
from jax.experimental.pallas import tpu as pltpu

_TOKENS = 16384
_D_IN = 1024
_D_OUT = 1024
_SLOTS = 64
_R = 16
_SCALING = 2.0
_BT = 1024


def _fused_body(map_ref, x_ref, wt_ref, b_ref, af_ref, bf_ref, out_ref):
    x = x_ref[...]
    base = jnp.dot(x, wt_ref[...], preferred_element_type=jnp.float32)
    h = jnp.dot(x.astype(jnp.bfloat16), af_ref[...],
                preferred_element_type=jnp.float32)
    m = map_ref[0, 0, :]
    col_slot = jax.lax.broadcasted_iota(jnp.int32, (_BT, _SLOTS * _R), 1) // _R
    hm = jnp.where(col_slot == m[:, None], h, 0.0).astype(jnp.bfloat16)
    lora = jnp.dot(hm, bf_ref[...], preferred_element_type=jnp.float32)
    out_ref[...] = base + b_ref[...] + _SCALING * lora


def kernel(x, adapter_mapping, W, b, lora_As, lora_Bs):
    n_blocks = _TOKENS // _BT
    wt = W.T
    a_flat = lora_As.transpose(1, 0, 2).reshape(_D_IN, _SLOTS * _R).astype(jnp.bfloat16)
    b_flat = lora_Bs.reshape(_SLOTS * _R, _D_OUT).astype(jnp.bfloat16)
    map3 = adapter_mapping.reshape(n_blocks, 1, _BT)
    b2 = b.reshape(1, _D_OUT)

    grid = (n_blocks,)
    out = pl.pallas_call(
        _fused_body,
        grid=grid,
        in_specs=[
            pl.BlockSpec((1, 1, _BT), lambda i: (i, 0, 0)),
            pl.BlockSpec((_BT, _D_IN), lambda i: (i, 0)),
            pl.BlockSpec((_D_IN, _D_OUT), lambda i: (0, 0)),
            pl.BlockSpec((1, _D_OUT), lambda i: (0, 0)),
            pl.BlockSpec((_D_IN, _SLOTS * _R), lambda i: (0, 0)),
            pl.BlockSpec((_SLOTS * _R, _D_OUT), lambda i: (0, 0)),
        ],
        out_specs=pl.BlockSpec((_BT, _D_OUT), lambda i: (i, 0)),
        out_shape=jax.ShapeDtypeStruct((_TOKENS, _D_OUT), jnp.float32),
        compiler_params=pltpu.CompilerParams(
            dimension_semantics=("arbitrary",),
        ),
    )(map3, x, wt, b2, a_flat, b_flat)
    return out
